# initial kernel scaffold (unmeasured)
import jax
import jax.numpy as jnp
from jax import lax
from jax.experimental import pallas as pl
from jax.experimental.pallas import tpu as pltpu

N_DEV = 4


def kernel(x, w_mat):
    m_per, k = x.shape
    k2, n = w_mat.shape
    assert k == k2
    n_per = n // N_DEV
    m_tot = m_per * N_DEV

    def body(x_ref, w_ref, out_ref, xb_ref, wpan_ref, send_ref, recv_ref,
             copy_sem, send_sems, recv_sems):
        me = lax.axis_index("i")

        barrier = pltpu.get_barrier_semaphore()
        for d in range(1, N_DEV):
            peer = lax.rem(me + d, N_DEV)
            pl.semaphore_signal(
                barrier, inc=1,
                device_id=(peer,), device_id_type=pl.DeviceIdType.MESH,
            )
        pl.semaphore_wait(barrier, N_DEV - 1)

        xb_ref[...] = x_ref[...].astype(jnp.bfloat16)

        def load_panel(t):
            cp = pltpu.make_async_copy(
                w_ref.at[:, pl.ds(t * n_per, n_per)], wpan_ref, copy_sem)
            cp.start()
            cp.wait()

        def block_for(t):
            load_panel(t)
            return jnp.dot(xb_ref[...], wpan_ref[...].astype(jnp.bfloat16),
                           preferred_element_type=jnp.float32)

        rdmas = []
        for d in range(1, N_DEV):
            tgt = lax.rem(me + d, N_DEV)
            send_ref[d - 1, :, :] = block_for(tgt).astype(jnp.bfloat16)
            rdma = pltpu.make_async_remote_copy(
                src_ref=send_ref.at[d - 1],
                dst_ref=recv_ref.at[d - 1],
                send_sem=send_sems.at[d - 1],
                recv_sem=recv_sems.at[d - 1],
                device_id=(tgt,),
                device_id_type=pl.DeviceIdType.MESH,
            )
            rdma.start()
            rdmas.append(rdma)

        out_ref[pl.ds(me * m_per, m_per), :] = block_for(me)

        for d in range(1, N_DEV):
            src = lax.rem(me + (N_DEV - d), N_DEV)
            rdmas[d - 1].wait_recv()
            out_ref[pl.ds(src * m_per, m_per), :] = (
                recv_ref[d - 1].astype(jnp.float32))

        for rdma in rdmas:
            rdma.wait_send()

    return pl.pallas_call(
        body,
        out_shape=jax.ShapeDtypeStruct((m_tot, n_per), jnp.float32),
        in_specs=[
            pl.BlockSpec(memory_space=pltpu.VMEM),
            pl.BlockSpec(memory_space=pltpu.ANY),
        ],
        out_specs=pl.BlockSpec(memory_space=pltpu.VMEM),
        scratch_shapes=[
            pltpu.VMEM((m_per, k), jnp.bfloat16),
            pltpu.VMEM((k, n_per), jnp.float32),
            pltpu.VMEM((N_DEV - 1, m_per, n_per), jnp.bfloat16),
            pltpu.VMEM((N_DEV - 1, m_per, n_per), jnp.bfloat16),
            pltpu.SemaphoreType.DMA,
            pltpu.SemaphoreType.DMA((N_DEV - 1,)),
            pltpu.SemaphoreType.DMA((N_DEV - 1,)),
        ],
        compiler_params=pltpu.CompilerParams(
            collective_id=0,
            vmem_limit_bytes=100 * 1024 * 1024,
        ),
    )(x, w_mat)


# baseline (device time: 60617 ns/iter reference)
import jax
import jax.numpy as jnp
from jax import lax
from jax.experimental import pallas as pl
from jax.experimental.pallas import tpu as pltpu

N_DEV = 4


def kernel(x, w_mat):
    m_per, k = x.shape
    k2, n = w_mat.shape
    assert k == k2
    n_per = n // N_DEV
    m_tot = m_per * N_DEV

    def body(x_ref, w_ref, out_ref, xb_ref, wpan_ref, send_ref, recv_ref,
             copy_sem, send_sems, recv_sems):
        me = lax.axis_index("i")

        barrier = pltpu.get_barrier_semaphore()
        for d in range(1, N_DEV):
            peer = lax.rem(me + d, N_DEV)
            pl.semaphore_signal(
                barrier, inc=1,
                device_id=(peer,), device_id_type=pl.DeviceIdType.MESH,
            )
        pl.semaphore_wait(barrier, N_DEV - 1)

        xb_ref[...] = x_ref[...].astype(jnp.bfloat16)

        def load_panel(t):
            cp = pltpu.make_async_copy(
                w_ref.at[:, pl.ds(t * n_per, n_per)], wpan_ref, copy_sem)
            cp.start()
            cp.wait()

        def block_for(t):
            load_panel(t)
            return jnp.dot(xb_ref[...], wpan_ref[...].astype(jnp.bfloat16),
                           preferred_element_type=jnp.float32)

        rdmas = []
        for d in range(1, N_DEV):
            tgt = lax.rem(me + d, N_DEV)
            send_ref[d - 1, :, :] = block_for(tgt).astype(jnp.bfloat16)
            rdma = pltpu.make_async_remote_copy(
                src_ref=send_ref.at[d - 1],
                dst_ref=recv_ref.at[d - 1],
                send_sem=send_sems.at[d - 1],
                recv_sem=recv_sems.at[d - 1],
                device_id=(tgt,),
                device_id_type=pl.DeviceIdType.MESH,
            )
            rdma.start()
            rdmas.append(rdma)

        out_ref[pl.ds(me * m_per, m_per), :] = block_for(me)

        for d in range(1, N_DEV):
            src = lax.rem(me + (N_DEV - d), N_DEV)
            rdmas[d - 1].wait_recv()
            out_ref[pl.ds(src * m_per, m_per), :] = (
                recv_ref[d - 1].astype(jnp.float32))

        for rdma in rdmas:
            rdma.wait_send()

    return pl.pallas_call(
        body,
        out_shape=jax.ShapeDtypeStruct((m_tot, n_per), jnp.float32),
        in_specs=[
            pl.BlockSpec(memory_space=pltpu.VMEM),
            pl.BlockSpec(memory_space=pl.ANY),
        ],
        out_specs=pl.BlockSpec(memory_space=pltpu.VMEM),
        scratch_shapes=[
            pltpu.VMEM((m_per, k), jnp.bfloat16),
            pltpu.VMEM((k, n_per), jnp.float32),
            pltpu.VMEM((N_DEV - 1, m_per, n_per), jnp.bfloat16),
            pltpu.VMEM((N_DEV - 1, m_per, n_per), jnp.bfloat16),
            pltpu.SemaphoreType.DMA,
            pltpu.SemaphoreType.DMA((N_DEV - 1,)),
            pltpu.SemaphoreType.DMA((N_DEV - 1,)),
        ],
        compiler_params=pltpu.CompilerParams(
            collective_id=0,
            vmem_limit_bytes=100 * 1024 * 1024,
        ),
    )(x, w_mat)


# device time: 57644 ns/iter; 1.0516x vs baseline; 1.0516x over previous
import jax
import jax.numpy as jnp
from jax import lax
from jax.experimental import pallas as pl
from jax.experimental.pallas import tpu as pltpu

N_DEV = 4


def kernel(x, w_mat):
    m_per, k = x.shape
    k2, n = w_mat.shape
    assert k == k2
    n_per = n // N_DEV
    n_half = n_per // 2
    m_tot = m_per * N_DEV

    d_order = (2, 1, 3, 0)
    chunks = [(d, h) for d in d_order for h in (0, 1)]

    def body(x_ref, w_ref, out_ref, xb_ref, wpan_ref, send_ref, recv_ref,
             copy_sems, send_sems, recv_sems):
        me = lax.axis_index("i")

        dmas = []

        def issue_dma(i):
            d, h = chunks[i]
            tgt = lax.rem(me + d, N_DEV)
            cp = pltpu.make_async_copy(
                w_ref.at[:, pl.ds(tgt * n_per + h * n_half, n_half)],
                wpan_ref.at[i % 2],
                copy_sems.at[i % 2])
            cp.start()
            dmas.append(cp)

        issue_dma(0)
        xb_ref[...] = x_ref[...].astype(jnp.bfloat16)

        barrier = pltpu.get_barrier_semaphore()
        for d in range(1, N_DEV):
            peer = lax.rem(me + d, N_DEV)
            pl.semaphore_signal(
                barrier, inc=1,
                device_id=(peer,), device_id_type=pl.DeviceIdType.MESH,
            )
        pl.semaphore_wait(barrier, N_DEV - 1)

        rdmas = {}
        for i, (d, h) in enumerate(chunks):
            if i + 1 < len(chunks):
                issue_dma(i + 1)
            dmas[i].wait()
            wb = wpan_ref[i % 2].astype(jnp.bfloat16)
            if d == 0:
                out_ref[pl.ds(me * m_per, m_per), pl.ds(h * n_half, n_half)] = (
                    jnp.dot(xb_ref[...], wb,
                            preferred_element_type=jnp.float32))
            else:
                tgt = lax.rem(me + d, N_DEV)
                send_ref[d - 1, :, pl.ds(h * n_half, n_half)] = jnp.dot(
                    xb_ref[...], wb,
                    preferred_element_type=jnp.float32).astype(jnp.bfloat16)
                rdma = pltpu.make_async_remote_copy(
                    src_ref=send_ref.at[d - 1, :, pl.ds(h * n_half, n_half)],
                    dst_ref=recv_ref.at[d - 1, :, pl.ds(h * n_half, n_half)],
                    send_sem=send_sems.at[d - 1, h],
                    recv_sem=recv_sems.at[d - 1, h],
                    device_id=(tgt,),
                    device_id_type=pl.DeviceIdType.MESH,
                )
                rdma.start()
                rdmas[(d, h)] = rdma

        for d, h in chunks:
            if d == 0:
                continue
            src = lax.rem(me + (N_DEV - d), N_DEV)
            rdmas[(d, h)].wait_recv()
            out_ref[pl.ds(src * m_per, m_per), pl.ds(h * n_half, n_half)] = (
                recv_ref[d - 1, :, pl.ds(h * n_half, n_half)]
                .astype(jnp.float32))

        for rdma in rdmas.values():
            rdma.wait_send()

    return pl.pallas_call(
        body,
        out_shape=jax.ShapeDtypeStruct((m_tot, n_per), jnp.float32),
        in_specs=[
            pl.BlockSpec(memory_space=pltpu.VMEM),
            pl.BlockSpec(memory_space=pl.ANY),
        ],
        out_specs=pl.BlockSpec(memory_space=pltpu.VMEM),
        scratch_shapes=[
            pltpu.VMEM((m_per, k), jnp.bfloat16),
            pltpu.VMEM((2, k, n_half), jnp.float32),
            pltpu.VMEM((N_DEV - 1, m_per, n_per), jnp.bfloat16),
            pltpu.VMEM((N_DEV - 1, m_per, n_per), jnp.bfloat16),
            pltpu.SemaphoreType.DMA((2,)),
            pltpu.SemaphoreType.DMA((N_DEV - 1, 2)),
            pltpu.SemaphoreType.DMA((N_DEV - 1, 2)),
        ],
        compiler_params=pltpu.CompilerParams(
            collective_id=0,
            vmem_limit_bytes=100 * 1024 * 1024,
        ),
    )(x, w_mat)
